# Initial kernel scaffold; baseline (speedup 1.0000x reference)
import jax
import jax.numpy as jnp
from jax.experimental import pallas as pl


def kernel(x, gate_w, gate_b, fc1_w, fc2_w, fc3_w):
    raise NotImplementedError("write your pallas kernel here")


# speedup vs baseline: 1.7621x; 1.7621x over previous
import functools

import jax
import jax.numpy as jnp
from jax import lax
from jax.experimental import pallas as pl
from jax.experimental.pallas import tpu as pltpu
from jax.experimental.pallas import tpu_sc as plsc

NUM_EXPERT = 8
EMBED_DIM = 768
FFN_PAD = 512
TILE = 256


@functools.lru_cache(maxsize=None)
def _sc_row_gather(n_rows_table, n_idx, dim):
    info = plsc.get_sparse_core_info()
    nc, ns = info.num_cores, info.num_subcores
    nw = nc * ns
    assert n_idx % nw == 0
    b_per_w = n_idx // nw
    chunk = 64
    assert b_per_w % chunk == 0
    n_chunks = b_per_w // chunk
    mesh = plsc.VectorSubcoreMesh(core_axis_name="c", subcore_axis_name="s")

    @functools.partial(
        pl.kernel,
        mesh=mesh,
        out_type=jax.ShapeDtypeStruct((n_idx, dim), jnp.float32),
        scratch_types=[
            pltpu.VMEM((chunk,), jnp.int32),
            pltpu.VMEM((chunk, dim), jnp.float32),
            pltpu.SemaphoreType.DMA,
        ],
    )
    def gather_k(table_hbm, idx_hbm, out_hbm, idx_v, rows_v, sem):
        wid = lax.axis_index("s") * nc + lax.axis_index("c")
        base = wid * b_per_w

        def body(c, carry):
            off = base + c * chunk
            pltpu.sync_copy(idx_hbm.at[pl.ds(off, chunk)], idx_v)
            pltpu.async_copy(table_hbm.at[idx_v], rows_v, sem).wait()
            pltpu.sync_copy(rows_v, out_hbm.at[pl.ds(off, chunk)])
            return carry

        lax.fori_loop(0, n_chunks, body, 0)

    return gather_k


def _gate_body(x_ref, gw_ref, gb_ref, eid_ref):
    x = x_ref[...]
    logits = lax.dot_general(
        x, gw_ref[...], (((1,), (1,)), ((), ())),
        preferred_element_type=jnp.float32,
    ) + gb_ref[...]
    m = jnp.max(logits, axis=1, keepdims=True)
    io = lax.broadcasted_iota(jnp.int32, logits.shape, 1)
    eid = jnp.min(jnp.where(logits >= m, io, NUM_EXPERT), axis=1)
    eid_ref[...] = eid.reshape(1, 1, -1)


def _gate_expert_ids(x_flat, gate_w, gate_b):
    n = x_flat.shape[0]
    blk = 1024
    grid = n // blk
    out = pl.pallas_call(
        _gate_body,
        grid=(grid,),
        in_specs=[
            pl.BlockSpec((blk, EMBED_DIM), lambda g: (g, 0)),
            pl.BlockSpec((NUM_EXPERT, EMBED_DIM), lambda g: (0, 0)),
            pl.BlockSpec((1, NUM_EXPERT), lambda g: (0, 0)),
        ],
        out_specs=pl.BlockSpec((1, 1, blk), lambda g: (g, 0, 0)),
        out_shape=jax.ShapeDtypeStruct((grid, 1, blk), jnp.int32),
    )(x_flat, gate_w, gate_b.reshape(1, NUM_EXPERT))
    return out.reshape(n)


def _moe_body(te_ref, x_ref, w1_ref, w2_ref, w3_ref, o_ref):
    x = x_ref[...]
    h1 = lax.dot_general(
        x, w1_ref[0], (((1,), (1,)), ((), ())),
        preferred_element_type=jnp.float32,
    )
    h2 = lax.dot_general(
        x, w2_ref[0], (((1,), (1,)), ((), ())),
        preferred_element_type=jnp.float32,
    )
    h = h1 * jax.nn.sigmoid(h1) * h2
    o_ref[...] = lax.dot_general(
        h, w3_ref[0], (((1,), (1,)), ((), ())),
        preferred_element_type=jnp.float32,
    )


def _grouped_ffn(tile_expert, x_sorted, fc1_p, fc2_p, fc3_p):
    p = x_sorted.shape[0]
    grid = p // TILE
    return pl.pallas_call(
        _moe_body,
        grid_spec=pltpu.PrefetchScalarGridSpec(
            num_scalar_prefetch=1,
            grid=(grid,),
            in_specs=[
                pl.BlockSpec((TILE, EMBED_DIM), lambda g, te: (g, 0)),
                pl.BlockSpec((1, FFN_PAD, EMBED_DIM), lambda g, te: (te[g], 0, 0)),
                pl.BlockSpec((1, FFN_PAD, EMBED_DIM), lambda g, te: (te[g], 0, 0)),
                pl.BlockSpec((1, EMBED_DIM, FFN_PAD), lambda g, te: (te[g], 0, 0)),
            ],
            out_specs=pl.BlockSpec((TILE, EMBED_DIM), lambda g, te: (g, 0)),
        ),
        out_shape=jax.ShapeDtypeStruct((p, EMBED_DIM), jnp.float32),
    )(tile_expert, x_sorted, fc1_p, fc2_p, fc3_p)


def kernel(x, gate_w, gate_b, fc1_w, fc2_w, fc3_w):
    b, s, d = x.shape
    n = b * s
    e = NUM_EXPERT
    p = n + e * TILE
    x_flat = x.reshape(n, d)

    f0 = fc1_w.shape[1]
    fc1_p = jnp.pad(fc1_w, ((0, 0), (0, FFN_PAD - f0), (0, 0)))
    fc2_p = jnp.pad(fc2_w, ((0, 0), (0, FFN_PAD - f0), (0, 0)))
    fc3_p = jnp.pad(fc3_w, ((0, 0), (0, 0), (0, FFN_PAD - f0)))

    eid = _gate_expert_ids(x_flat, gate_w, gate_b)

    onehot = (eid[:, None] == jnp.arange(e, dtype=jnp.int32)[None, :]).astype(jnp.int32)
    csum = jnp.cumsum(onehot, axis=0)
    counts = csum[-1]
    rank = jnp.take_along_axis(csum - onehot, eid[:, None], axis=1)[:, 0]
    aligned = ((counts + TILE - 1) // TILE) * TILE
    starts = jnp.concatenate(
        [jnp.zeros((1,), jnp.int32), jnp.cumsum(aligned)[:-1].astype(jnp.int32)]
    )
    inv_perm = starts[eid] + rank
    perm = jnp.zeros((p,), jnp.int32).at[inv_perm].set(
        jnp.arange(n, dtype=jnp.int32)
    )
    tile_base = jnp.arange(p // TILE, dtype=jnp.int32) * TILE
    tile_expert = jnp.clip(
        jnp.searchsorted(starts, tile_base, side="right") - 1, 0, e - 1
    ).astype(jnp.int32)

    x_sorted = _sc_row_gather(n, p, d)(x_flat, perm)
    out_sorted = _grouped_ffn(tile_expert, x_sorted, fc1_p, fc2_p, fc3_p)
    out_flat = _sc_row_gather(p, n, d)(out_sorted, inv_perm)
    return out_flat.reshape(b, s, d)


# speedup vs baseline: 3.2823x; 1.8627x over previous
import functools

import jax
import jax.numpy as jnp
from jax import lax
from jax.experimental import pallas as pl
from jax.experimental.pallas import tpu as pltpu
from jax.experimental.pallas import tpu_sc as plsc

NUM_EXPERT = 8
EMBED_DIM = 768
FFN_DIM = 469
TILE = 256


@functools.lru_cache(maxsize=None)
def _sc_row_gather(n_rows_table, n_idx, dim):
    info = plsc.get_sparse_core_info()
    nc, ns = info.num_cores, info.num_subcores
    nw = nc * ns
    assert n_idx % nw == 0
    b_per_w = n_idx // nw
    chunk = 64
    assert b_per_w % chunk == 0
    n_chunks = b_per_w // chunk
    mesh = plsc.VectorSubcoreMesh(core_axis_name="c", subcore_axis_name="s")

    @functools.partial(
        pl.kernel,
        mesh=mesh,
        out_type=jax.ShapeDtypeStruct((n_idx, dim), jnp.float32),
        scratch_types=[
            pltpu.VMEM((chunk,), jnp.int32),
            pltpu.VMEM((chunk, dim), jnp.float32),
            pltpu.SemaphoreType.DMA,
        ],
    )
    def gather_k(table_hbm, idx_hbm, out_hbm, idx_v, rows_v, sem):
        wid = lax.axis_index("s") * nc + lax.axis_index("c")
        base = wid * b_per_w

        def body(c, carry):
            off = base + c * chunk
            pltpu.sync_copy(idx_hbm.at[pl.ds(off, chunk)], idx_v)
            pltpu.async_copy(table_hbm.at[idx_v], rows_v, sem).wait()
            pltpu.sync_copy(rows_v, out_hbm.at[pl.ds(off, chunk)])
            return carry

        lax.fori_loop(0, n_chunks, body, 0)

    return gather_k


@functools.lru_cache(maxsize=None)
def _sc_row_scatter(n_rows_out, n_idx, dim):
    info = plsc.get_sparse_core_info()
    nc, ns = info.num_cores, info.num_subcores
    nw = nc * ns
    assert n_idx % nw == 0
    b_per_w = n_idx // nw
    chunk = 64
    assert b_per_w % chunk == 0
    n_chunks = b_per_w // chunk
    mesh = plsc.VectorSubcoreMesh(core_axis_name="c", subcore_axis_name="s")

    @functools.partial(
        pl.kernel,
        mesh=mesh,
        out_type=jax.ShapeDtypeStruct((n_rows_out, dim), jnp.float32),
        scratch_types=[
            pltpu.VMEM((chunk,), jnp.int32),
            pltpu.VMEM((chunk, dim), jnp.float32),
            pltpu.SemaphoreType.DMA,
        ],
    )
    def scatter_k(rows_hbm, idx_hbm, out_hbm, idx_v, rows_v, sem):
        wid = lax.axis_index("s") * nc + lax.axis_index("c")
        base = wid * b_per_w

        def body(c, carry):
            off = base + c * chunk
            pltpu.sync_copy(idx_hbm.at[pl.ds(off, chunk)], idx_v)
            pltpu.sync_copy(rows_hbm.at[pl.ds(off, chunk)], rows_v)
            pltpu.async_copy(rows_v, out_hbm.at[idx_v], sem).wait()
            return carry

        lax.fori_loop(0, n_chunks, body, 0)

    return scatter_k


def _gate_body(x_ref, gw_ref, gb_ref, eid_ref):
    x = x_ref[...]
    logits = lax.dot_general(
        x, gw_ref[...], (((1,), (1,)), ((), ())),
        preferred_element_type=jnp.float32,
    ) + gb_ref[...]
    m = jnp.max(logits, axis=1, keepdims=True)
    io = lax.broadcasted_iota(jnp.int32, logits.shape, 1)
    eid = jnp.min(jnp.where(logits >= m, io, NUM_EXPERT), axis=1)
    eid_ref[...] = eid.reshape(1, 1, -1)


def _gate_expert_ids(x_flat, gate_w, gate_b):
    n = x_flat.shape[0]
    blk = 1024
    grid = n // blk
    out = pl.pallas_call(
        _gate_body,
        grid=(grid,),
        in_specs=[
            pl.BlockSpec((blk, EMBED_DIM), lambda g: (g, 0)),
            pl.BlockSpec((NUM_EXPERT, EMBED_DIM), lambda g: (0, 0)),
            pl.BlockSpec((1, NUM_EXPERT), lambda g: (0, 0)),
        ],
        out_specs=pl.BlockSpec((1, 1, blk), lambda g: (g, 0, 0)),
        out_shape=jax.ShapeDtypeStruct((grid, 1, blk), jnp.int32),
    )(x_flat, gate_w, gate_b.reshape(1, NUM_EXPERT))
    return out.reshape(n)


def _moe_body(te_ref, x_ref, w1_ref, w2_ref, w3_ref, o_ref):
    x = x_ref[...]
    h1 = lax.dot_general(
        x, w1_ref[0], (((1,), (1,)), ((), ())),
        preferred_element_type=jnp.float32,
    )
    h2 = lax.dot_general(
        x, w2_ref[0], (((1,), (1,)), ((), ())),
        preferred_element_type=jnp.float32,
    )
    h = h1 * jax.nn.sigmoid(h1) * h2
    o_ref[...] = lax.dot_general(
        h, w3_ref[0], (((1,), (1,)), ((), ())),
        preferred_element_type=jnp.float32,
    )


def _grouped_ffn(tile_expert, x_sorted, fc1_w, fc2_w, fc3_w):
    p = x_sorted.shape[0]
    grid = p // TILE
    return pl.pallas_call(
        _moe_body,
        grid_spec=pltpu.PrefetchScalarGridSpec(
            num_scalar_prefetch=1,
            grid=(grid,),
            in_specs=[
                pl.BlockSpec((TILE, EMBED_DIM), lambda g, te: (g, 0)),
                pl.BlockSpec((1, FFN_DIM, EMBED_DIM), lambda g, te: (te[g], 0, 0)),
                pl.BlockSpec((1, FFN_DIM, EMBED_DIM), lambda g, te: (te[g], 0, 0)),
                pl.BlockSpec((1, EMBED_DIM, FFN_DIM), lambda g, te: (te[g], 0, 0)),
            ],
            out_specs=pl.BlockSpec((TILE, EMBED_DIM), lambda g, te: (g, 0)),
        ),
        out_shape=jax.ShapeDtypeStruct((p, EMBED_DIM), jnp.float32),
    )(tile_expert, x_sorted, fc1_w, fc2_w, fc3_w)


def kernel(x, gate_w, gate_b, fc1_w, fc2_w, fc3_w):
    b, s, d = x.shape
    n = b * s
    e = NUM_EXPERT
    p = n + e * TILE
    x_flat = x.reshape(n, d)

    eid = _gate_expert_ids(x_flat, gate_w, gate_b)

    onehot = (eid[:, None] == jnp.arange(e, dtype=jnp.int32)[None, :]).astype(jnp.int32)
    csum = jnp.cumsum(onehot, axis=0)
    counts = csum[-1]
    rank = jnp.sum(csum * onehot, axis=1) - 1
    aligned = ((counts + TILE - 1) // TILE) * TILE
    starts = jnp.concatenate(
        [jnp.zeros((1,), jnp.int32), jnp.cumsum(aligned)[:-1].astype(jnp.int32)]
    )
    start_per_tok = jnp.sum(starts[None, :] * onehot, axis=1)
    inv_perm = start_per_tok + rank
    tile_base = jnp.arange(p // TILE, dtype=jnp.int32) * TILE
    tile_expert = (
        jnp.sum((tile_base[:, None] >= starts[None, :]).astype(jnp.int32), axis=1) - 1
    )
    tile_expert = jnp.clip(tile_expert, 0, e - 1).astype(jnp.int32)

    x_sorted = _sc_row_scatter(p, n, d)(x_flat, inv_perm)
    out_sorted = _grouped_ffn(tile_expert, x_sorted, fc1_w, fc2_w, fc3_w)
    out_flat = _sc_row_gather(p, n, d)(out_sorted, inv_perm)
    return out_flat.reshape(b, s, d)
